# Initial kernel scaffold; baseline (speedup 1.0000x reference)
import jax
import jax.numpy as jnp
from jax.experimental import pallas as pl


def kernel(hidden_states, router_w, correction_bias, w_gate, w_up, w_down, num_global_tokens, max_num_tokens_per_gpu):
    raise NotImplementedError("write your pallas kernel here")


# speedup vs baseline: 1.2262x; 1.2262x over previous
import functools

import jax
import jax.numpy as jnp
from jax.experimental import pallas as pl
from jax.experimental.pallas import tpu as pltpu

T = 2048
D = 1024
FF = 1024
E = 8
TOP_K = 2
BM = 512


def _moe_kernel(x_ref, rw_ref, bias_ref, wg_ref, wu_ref, wd_ref, out_ref,
                combine_ref):
    e = pl.program_id(1)
    x = x_ref[...]

    @pl.when(e == 0)
    def _router():
        logits = jax.lax.dot_general(
            x, rw_ref[...], (((1,), (1,)), ((), ())),
            preferred_element_type=jnp.float32,
            precision=jax.lax.Precision.DEFAULT)
        m = jnp.max(logits, axis=-1, keepdims=True)
        ex = jnp.exp(logits - m)
        scores = ex / jnp.sum(ex, axis=-1, keepdims=True)
        sel = scores + bias_ref[...]
        lane = jax.lax.broadcasted_iota(jnp.int32, (BM, E), 1)
        BIG = jnp.int32(2 * E)
        NEG = jnp.float32(-1e30)
        m1 = jnp.max(sel, axis=-1, keepdims=True)
        i1 = jnp.min(jnp.where(sel == m1, lane, BIG), axis=-1, keepdims=True)
        oh1 = lane == i1
        sel2 = jnp.where(oh1, NEG, sel)
        m2 = jnp.max(sel2, axis=-1, keepdims=True)
        i2 = jnp.min(jnp.where(sel2 == m2, lane, BIG), axis=-1, keepdims=True)
        oh2 = lane == i2
        combine_ref[...] = jnp.where(oh1 | oh2, scores, 0.0)

    xb = x.astype(jnp.bfloat16)
    g = jax.lax.dot_general(xb, wg_ref[0], (((1,), (1,)), ((), ())),
                            preferred_element_type=jnp.float32)
    u = jax.lax.dot_general(xb, wu_ref[0], (((1,), (1,)), ((), ())),
                            preferred_element_type=jnp.float32)
    h = (g * jax.lax.logistic(g)) * u
    y = jax.lax.dot_general(h.astype(jnp.bfloat16), wd_ref[0],
                            (((1,), (0,)), ((), ())),
                            preferred_element_type=jnp.float32)
    lane = jax.lax.broadcasted_iota(jnp.int32, (BM, E), 1)
    w_e = jnp.sum(jnp.where(lane == e, combine_ref[...], 0.0), axis=-1,
                  keepdims=True)
    wy = w_e * y

    @pl.when(e == 0)
    def _init():
        out_ref[...] = wy

    @pl.when(e != 0)
    def _acc():
        out_ref[...] += wy


def kernel(hidden_states, router_w, correction_bias, w_gate, w_up, w_down,
           num_global_tokens, max_num_tokens_per_gpu):
    x = hidden_states.astype(jnp.float32)
    wg = w_gate.astype(jnp.bfloat16)
    wu = w_up.astype(jnp.bfloat16)
    wd = jnp.swapaxes(w_down, 1, 2).astype(jnp.bfloat16)
    bias = correction_bias.reshape(1, E).astype(jnp.float32)

    grid = (T // BM, E)
    out = pl.pallas_call(
        _moe_kernel,
        grid=grid,
        in_specs=[
            pl.BlockSpec((BM, D), lambda i, e: (i, 0)),
            pl.BlockSpec((E, D), lambda i, e: (0, 0)),
            pl.BlockSpec((1, E), lambda i, e: (0, 0)),
            pl.BlockSpec((1, FF, D), lambda i, e: (e, 0, 0)),
            pl.BlockSpec((1, FF, D), lambda i, e: (e, 0, 0)),
            pl.BlockSpec((1, FF, D), lambda i, e: (e, 0, 0)),
        ],
        out_specs=pl.BlockSpec((BM, D), lambda i, e: (i, 0)),
        out_shape=jax.ShapeDtypeStruct((T, D), jnp.float32),
        scratch_shapes=[pltpu.VMEM((BM, E), jnp.float32)],
    )(x, router_w.astype(jnp.float32), bias, wg, wu, wd)
    return out
